# baseline (speedup 1.0000x reference)
import functools
import math

import jax
import jax.numpy as jnp
from jax import lax
from jax.experimental import pallas as pl
from jax.experimental.pallas import tpu as pltpu
from jax.experimental.pallas import tpu_sc as plsc

_EMB = 32
_N_TOK = 16384 * 50
_NW = 32
_PER_W = _N_TOK // _NW
_CHUNK = 128
_N_CHUNK = _PER_W // _CHUNK
_SCALE = math.sqrt(_EMB)

_mesh = plsc.VectorSubcoreMesh(core_axis_name="c", subcore_axis_name="s")


@functools.partial(
    pl.kernel,
    mesh=_mesh,
    compiler_params=pltpu.CompilerParams(use_tc_tiling_on_sc=False),
    out_type=jax.ShapeDtypeStruct((_N_TOK, _EMB), jnp.float32),
    scratch_types=[
        pltpu.VMEM((_N_CHUNK, _CHUNK), jnp.int32),
        pltpu.VMEM((_CHUNK, _EMB), jnp.float32),
        pltpu.SemaphoreType.DMA,
    ],
)
def _emb_lookup(tok_hbm, table_hbm, out_hbm, idx_v, buf, sem):
    wid = lax.axis_index("s") * 2 + lax.axis_index("c")
    base = wid * _PER_W
    pltpu.sync_copy(tok_hbm.at[wid], idx_v)

    def chunk_body(j, carry):
        pltpu.async_copy(table_hbm.at[idx_v.at[j]], buf, sem).wait()

        def scale_body(i, c):
            buf[i, pl.ds(0, 16)] = buf[i, pl.ds(0, 16)] * _SCALE
            buf[i, pl.ds(16, 16)] = buf[i, pl.ds(16, 16)] * _SCALE
            return c

        lax.fori_loop(0, _CHUNK, scale_body, 0)
        pltpu.sync_copy(buf, out_hbm.at[pl.ds(base + j * _CHUNK, _CHUNK)])
        return carry

    lax.fori_loop(0, _N_CHUNK, chunk_body, 0)


def kernel(tokens, table):
    tok = tokens.astype(jnp.int32).reshape(_NW, _N_CHUNK, _CHUNK)
    out = _emb_lookup(tok, table)
    return out.reshape(tokens.shape[0], tokens.shape[1], _EMB)


# speedup vs baseline: 1.1555x; 1.1555x over previous
import functools
import math

import jax
import jax.numpy as jnp
from jax import lax
from jax.experimental import pallas as pl
from jax.experimental.pallas import tpu as pltpu
from jax.experimental.pallas import tpu_sc as plsc

_EMB = 32
_N_TOK = 16384 * 50
_NW = 32
_PER_W = _N_TOK // _NW
_CHUNK = 128
_N_CHUNK = _PER_W // _CHUNK
_SUP = 4
_SROWS = _SUP * _CHUNK
_N_SUP = _PER_W // _SROWS
_SCALE = math.sqrt(_EMB)

_mesh = plsc.VectorSubcoreMesh(core_axis_name="c", subcore_axis_name="s")


@functools.partial(
    pl.kernel,
    mesh=_mesh,
    compiler_params=pltpu.CompilerParams(use_tc_tiling_on_sc=False),
    out_type=jax.ShapeDtypeStruct((_N_TOK, _EMB), jnp.float32),
    scratch_types=[
        pltpu.VMEM((_N_CHUNK, _CHUNK), jnp.int32),
        pltpu.VMEM((_SROWS, _EMB), jnp.float32),
        pltpu.VMEM((_SROWS, _EMB), jnp.float32),
        pltpu.VMEM((_SROWS, _EMB), jnp.float32),
        pltpu.SemaphoreType.DMA,
        pltpu.SemaphoreType.DMA,
        pltpu.SemaphoreType.DMA,
        pltpu.SemaphoreType.DMA,
        pltpu.SemaphoreType.DMA,
        pltpu.SemaphoreType.DMA,
    ],
)
def _emb_lookup(tok_hbm, table_hbm, out_hbm, idx_v, b0, b1, b2,
                g0, g1, g2, o0, o1, o2):
    wid = lax.axis_index("s") * 2 + lax.axis_index("c")
    base = wid * _PER_W
    pltpu.sync_copy(tok_hbm.at[wid], idx_v)

    bufs = ((b0, g0, o0), (b1, g1, o1), (b2, g2, o2))

    def issue_g(s, buf, gsem):
        for k in range(_SUP):
            pltpu.make_async_copy(
                table_hbm.at[idx_v.at[s * _SUP + k]],
                buf.at[pl.ds(k * _CHUNK, _CHUNK)],
                gsem,
            ).start()

    def wait_g(buf, gsem):
        for k in range(_SUP):
            pltpu.make_async_copy(
                table_hbm.at[idx_v.at[k]],
                buf.at[pl.ds(k * _CHUNK, _CHUNK)],
                gsem,
            ).wait()

    def issue_o(s, buf, osem):
        pltpu.make_async_copy(
            buf, out_hbm.at[pl.ds(base + s * _SROWS, _SROWS)], osem,
        ).start()

    def wait_o(buf, osem):
        pltpu.make_async_copy(
            buf, out_hbm.at[pl.ds(base, _SROWS)], osem,
        ).wait()

    def scale(buf):
        def body(i, c):
            for r in range(8):
                row = i * 8 + r
                buf[row, pl.ds(0, 16)] = buf[row, pl.ds(0, 16)] * _SCALE
                buf[row, pl.ds(16, 16)] = buf[row, pl.ds(16, 16)] * _SCALE
            return c
        lax.fori_loop(0, _SROWS // 8, body, 0)

    def stage(s, b, prefetch, first):
        buf, gsem, osem = bufs[b]
        wait_g(buf, gsem)
        scale(buf)
        issue_o(s, buf, osem)
        if prefetch:
            nbuf, ngsem, nosem = bufs[(b + 2) % 3]
            if not first:
                wait_o(nbuf, nosem)
            issue_g(s + 2, nbuf, ngsem)

    issue_g(0, b0, g0)
    issue_g(1, b1, g1)
    stage(0, 0, True, True)
    stage(1, 1, True, False)
    stage(2, 2, True, False)

    def triple(t, c):
        for b in range(3):
            s = t * 3 + b
            stage(s, b, True, False)
        return c
    lax.fori_loop(1, (_N_SUP - 2) // 3, triple, 0)

    stage(_N_SUP - 2, (_N_SUP - 2) % 3, False, False)
    stage(_N_SUP - 1, (_N_SUP - 1) % 3, False, False)
    for s in (_N_SUP - 3, _N_SUP - 2, _N_SUP - 1):
        buf, _, osem = bufs[s % 3]
        wait_o(buf, osem)


def kernel(tokens, table):
    tok = tokens.astype(jnp.int32).reshape(_NW, _N_CHUNK, _CHUNK)
    out = _emb_lookup(tok, table)
    return out.reshape(tokens.shape[0], tokens.shape[1], _EMB)


# speedup vs baseline: 1.8789x; 1.6260x over previous
import functools
import math

import jax
import jax.numpy as jnp
from jax import lax
from jax.experimental import pallas as pl
from jax.experimental.pallas import tpu as pltpu
from jax.experimental.pallas import tpu_sc as plsc

_EMB = 32
_B = 16384
_L = 50
_NW = 32
_BW = _B // _NW
_CHUNK = 128
_NCH = _BW // _CHUNK
_SCALE = math.sqrt(_EMB)

_mesh = plsc.VectorSubcoreMesh(core_axis_name="c", subcore_axis_name="s")


@functools.partial(
    pl.kernel,
    mesh=_mesh,
    compiler_params=pltpu.CompilerParams(use_tc_tiling_on_sc=False),
    out_type=jax.ShapeDtypeStruct((_B, _L, _EMB), jnp.float32),
    scratch_types=[
        pltpu.VMEM((_L, _NCH, _CHUNK), jnp.int32),
        pltpu.VMEM((_BW, _EMB), jnp.float32),
        pltpu.VMEM((_BW, _EMB), jnp.float32),
        pltpu.VMEM((_BW, _EMB), jnp.float32),
        pltpu.SemaphoreType.DMA,
        pltpu.SemaphoreType.DMA,
        pltpu.SemaphoreType.DMA,
        pltpu.SemaphoreType.DMA,
        pltpu.SemaphoreType.DMA,
        pltpu.SemaphoreType.DMA,
    ],
)
def _emb_lookup(tokt_hbm, table_hbm, out_hbm, tok_v, b0_, b1_, b2_,
                g0, g1, g2, o0, o1, o2):
    wid = lax.axis_index("s") * 2 + lax.axis_index("c")
    base = wid * _BW
    for k in range(_NCH):
        pltpu.sync_copy(
            tokt_hbm.at[:, pl.ds(base + k * _CHUNK, _CHUNK)],
            tok_v.at[:, k],
        )

    bufs = ((b0_, g0, o0), (b1_, g1, o1), (b2_, g2, o2))

    def issue_g(l, buf, gsem):
        for k in range(_NCH):
            pltpu.make_async_copy(
                table_hbm.at[tok_v.at[l, k]],
                buf.at[pl.ds(k * _CHUNK, _CHUNK)],
                gsem,
            ).start()

    def wait_g(buf, gsem):
        for k in range(_NCH):
            pltpu.make_async_copy(
                table_hbm.at[tok_v.at[0, k]],
                buf.at[pl.ds(k * _CHUNK, _CHUNK)],
                gsem,
            ).wait()

    def issue_o(l, buf, osem):
        pltpu.make_async_copy(
            buf, out_hbm.at[pl.ds(base, _BW), l], osem,
        ).start()

    def wait_o(buf, osem):
        pltpu.make_async_copy(
            buf, out_hbm.at[pl.ds(base, _BW), 0], osem,
        ).wait()

    def scale(buf):
        def body(i, c):
            for r in range(8):
                row = i * 8 + r
                buf[row, pl.ds(0, 16)] = buf[row, pl.ds(0, 16)] * _SCALE
                buf[row, pl.ds(16, 16)] = buf[row, pl.ds(16, 16)] * _SCALE
            return c
        lax.fori_loop(0, _BW // 8, body, 0)

    def stage(l, b, prefetch, first):
        buf, gsem, osem = bufs[b]
        wait_g(buf, gsem)
        scale(buf)
        issue_o(l, buf, osem)
        if prefetch:
            nbuf, ngsem, nosem = bufs[(b + 2) % 3]
            if not first:
                wait_o(nbuf, nosem)
            issue_g(l + 2, nbuf, ngsem)

    issue_g(0, b0_, g0)
    issue_g(1, b1_, g1)
    stage(0, 0, True, True)
    stage(1, 1, True, False)
    stage(2, 2, True, False)

    def triple(t, c):
        for b in range(3):
            stage(t * 3 + b, b, True, False)
        return c
    lax.fori_loop(1, (_L - 2) // 3, triple, 0)

    stage(_L - 2, (_L - 2) % 3, False, False)
    stage(_L - 1, (_L - 1) % 3, False, False)
    for l in (_L - 3, _L - 2, _L - 1):
        buf, _, osem = bufs[l % 3]
        wait_o(buf, osem)


def kernel(tokens, table):
    return _emb_lookup(tokens.T.astype(jnp.int32), table)
